# baseline (speedup 1.0000x reference)
import functools
import math

import jax
import jax.numpy as jnp
from jax import lax
from jax.experimental import pallas as pl
from jax.experimental.pallas import tpu as pltpu
from jax.experimental.pallas import tpu_sc as plsc

_TOPK = 128
_SEQ = 4096
_DK = 64
_INT_MIN = -2147483648
_DUMP = 144


def _importance_body(q_ref, lidx_ref, gp_ref):
    x = q_ref[0]
    mean = jnp.mean(x, axis=-1)
    xc = x - mean[:, None]
    var = jnp.sum(xc * xc, axis=-1) * (1.0 / (_DK - 1))
    imp = mean + jnp.sqrt(var)

    u = lax.bitcast_convert_type(imp, jnp.int32)
    ms = jnp.where(u >= 0, u, u ^ jnp.int32(0x7FFFFFFF))
    m2 = ms.reshape(_SEQ // 128, 128)

    cnt0 = jnp.sum((m2 >= 0).astype(jnp.int32))
    prefix0 = jnp.where(cnt0 >= _TOPK, jnp.int32(0), jnp.int32(_INT_MIN))

    def bit_body(b, prefix):
        bit = lax.shift_left(jnp.int32(1), jnp.int32(30) - b)
        cand = prefix | bit
        cnt = jnp.sum((m2 >= cand).astype(jnp.int32))
        return jnp.where(cnt >= _TOPK, cand, prefix)

    t = lax.fori_loop(0, 31, bit_body, prefix0)

    gt = m2 > t
    eq = m2 == t
    gti = gt.astype(jnp.float32)
    eqi = eq.astype(jnp.float32)
    nrow = _SEQ // 128
    u128 = (lax.broadcasted_iota(jnp.int32, (128, 128), 0)
            < lax.broadcasted_iota(jnp.int32, (128, 128), 1)).astype(jnp.float32)
    l32 = (lax.broadcasted_iota(jnp.int32, (nrow, nrow), 0)
           > lax.broadcasted_iota(jnp.int32, (nrow, nrow), 1)).astype(jnp.float32)

    def ex_prefix(fi):
        lane = lax.dot_general(fi, u128, (((1,), (0,)), ((), ())),
                               preferred_element_type=jnp.float32)
        rowsum = jnp.sum(fi, axis=1, keepdims=True)
        row = lax.dot_general(l32, rowsum, (((1,), (0,)), ((), ())),
                              preferred_element_type=jnp.float32)
        return (lane + row).astype(jnp.int32)

    pgt = ex_prefix(gti)
    peq = ex_prefix(eqi)
    c_gt = jnp.sum(gti).astype(jnp.int32)
    r = jnp.int32(_TOPK) - c_gt
    dest = jnp.where(gt, pgt,
                     jnp.where(eq & (peq < r), c_gt + peq, jnp.int32(_TOPK)))

    tio = lax.broadcasted_iota(jnp.int32, (nrow, 128, _TOPK), 2)
    e2 = (dest[:, :, None] == tio).astype(jnp.int32)
    flatf = (lax.broadcasted_iota(jnp.int32, (nrow, 128), 0) * 128
             + lax.broadcasted_iota(jnp.int32, (nrow, 128), 1))
    contrib = e2 * flatf[:, :, None]
    inv = jnp.sum(jnp.sum(contrib, axis=0), axis=0)
    lidx_ref[0, 0] = inv
    gp_ref[0, 0] = ((inv + jnp.int32(_SEQ) * pl.program_id(0))
                    >> 1)


def _run_importance(q3):
    n = q3.shape[0]
    return pl.pallas_call(
        _importance_body,
        grid=(n,),
        in_specs=[pl.BlockSpec((1, _SEQ, _DK), lambda i: (i, 0, 0))],
        out_specs=[
            pl.BlockSpec((1, 1, _TOPK), lambda i: (i, 0, 0)),
            pl.BlockSpec((1, 1, _TOPK), lambda i: (i, 0, 0)),
        ],
        out_shape=[
            jax.ShapeDtypeStruct((n, 1, _TOPK), jnp.int32),
            jax.ShapeDtypeStruct((n, 1, _TOPK), jnp.int32),
        ],
    )(q3)


def _sc_body(gidx_hbm, q_hbm, k_hbm, v_hbm,
             qs_hbm, ks_hbm, vs_hbm,
             gidx_v, qs_v, ks_v, vs_v, sem):
    p = lax.axis_index("s") * 2 + lax.axis_index("c")

    pltpu.sync_copy(gidx_hbm.at[p], gidx_v)

    cpq = pltpu.async_copy(q_hbm.at[gidx_v], qs_v, sem)
    cpk = pltpu.async_copy(k_hbm.at[gidx_v], ks_v, sem)
    cpv = pltpu.async_copy(v_hbm.at[gidx_v], vs_v, sem)
    cpq.wait()
    cpk.wait()
    cpv.wait()

    pltpu.sync_copy(qs_v, qs_hbm.at[p])
    pltpu.sync_copy(ks_v, ks_hbm.at[p])
    pltpu.sync_copy(vs_v, vs_hbm.at[p])


def _run_select_gather(gidx2, q2, k2, v2):
    n = gidx2.shape[0]
    mesh = plsc.VectorSubcoreMesh(core_axis_name="c", subcore_axis_name="s")
    f = functools.partial(
        pl.kernel,
        mesh=mesh,
        out_type=[
            jax.ShapeDtypeStruct((n, _TOPK, 2 * _DK), jnp.float32),
            jax.ShapeDtypeStruct((n, _TOPK, 2 * _DK), jnp.float32),
            jax.ShapeDtypeStruct((n, _TOPK, 2 * _DK), jnp.float32),
        ],
        scratch_types=[
            pltpu.VMEM((_TOPK,), jnp.int32),
            pltpu.VMEM((_TOPK, 2 * _DK), jnp.float32),
            pltpu.VMEM((_TOPK, 2 * _DK), jnp.float32),
            pltpu.VMEM((_TOPK, 2 * _DK), jnp.float32),
            pltpu.SemaphoreType.DMA,
        ],
    )(_sc_body)
    return f(gidx2, q2, k2, v2)


def _attention_body(qs_ref, ks_ref, vs_ref, idx_ref, out_ref):
    local = idx_ref[0]
    par = (local & 1)[0][:, None] == 1

    def pick(ref):
        full = ref[0]
        return jnp.where(par, full[:, _DK:], full[:, :_DK])

    qb = pick(qs_ref)
    kb = pick(ks_ref)
    vb = pick(vs_ref)
    s = lax.dot_general(qb, kb, (((1,), (1,)), ((), ())),
                        preferred_element_type=jnp.float32)
    s = s * (1.0 / math.sqrt(_DK))
    mx = jnp.max(s, axis=-1, keepdims=True)
    e = jnp.exp(s - mx)
    w = e / jnp.sum(e, axis=-1, keepdims=True)
    att = lax.dot_general(w, vb, (((1,), (0,)), ((), ())),
                          preferred_element_type=jnp.float32)
    rows = lax.broadcasted_iota(jnp.int32, (_SEQ, _TOPK), 0)
    p = (rows == local).astype(jnp.float32)
    out_ref[0] = lax.dot_general(p, att, (((1,), (0,)), ((), ())),
                                 preferred_element_type=jnp.float32)


def _run_attention(qs, ks, vs, idx3):
    n = qs.shape[0]
    return pl.pallas_call(
        _attention_body,
        grid=(n,),
        in_specs=[
            pl.BlockSpec((1, _TOPK, 2 * _DK), lambda i: (i, 0, 0)),
            pl.BlockSpec((1, _TOPK, 2 * _DK), lambda i: (i, 0, 0)),
            pl.BlockSpec((1, _TOPK, 2 * _DK), lambda i: (i, 0, 0)),
            pl.BlockSpec((1, 1, _TOPK), lambda i: (i, 0, 0)),
        ],
        out_specs=pl.BlockSpec((1, _SEQ, _DK), lambda i: (i, 0, 0)),
        out_shape=jax.ShapeDtypeStruct((n, _SEQ, _DK), jnp.float32),
    )(qs, ks, vs, idx3)


def kernel(q, k, v):
    B, H, S, D = q.shape
    n = B * H
    q3 = q.reshape(n, S, D)
    k3 = k.reshape(n, S, D)
    v3 = v.reshape(n, S, D)

    lidx, gpair = _run_importance(q3)
    gpair2 = gpair.reshape(n, _TOPK)

    qs, ks, vs = _run_select_gather(
        gpair2, q3.reshape(n * S // 2, 2 * D), k3.reshape(n * S // 2, 2 * D),
        v3.reshape(n * S // 2, 2 * D))

    out = _run_attention(qs, ks, vs, lidx)
    hw = int(math.sqrt(S))
    return out.reshape(B, -1, hw, hw)


# speedup vs baseline: 2.1410x; 2.1410x over previous
import functools
import math

import jax
import jax.numpy as jnp
from jax import lax
from jax.experimental import pallas as pl
from jax.experimental.pallas import tpu as pltpu
from jax.experimental.pallas import tpu_sc as plsc

_TOPK = 128
_SEQ = 4096
_DK = 64
_INT_MIN = -2147483648
_DUMP = 144


def _importance_body(q_ref, ms_ref):
    x = q_ref[0]
    mean = jnp.mean(x, axis=-1)
    xc = x - mean[:, None]
    var = jnp.sum(xc * xc, axis=-1) * (1.0 / (_DK - 1))
    imp = mean + jnp.sqrt(var)

    u = lax.bitcast_convert_type(imp, jnp.int32)
    ms = jnp.where(u >= 0, u, u ^ jnp.int32(0x7FFFFFFF))
    ms_ref[0] = ms.reshape(_SEQ // 128, 128)


def _run_importance(q3):
    n = q3.shape[0]
    return pl.pallas_call(
        _importance_body,
        grid=(n,),
        in_specs=[pl.BlockSpec((1, _SEQ, _DK), lambda i: (i, 0, 0))],
        out_specs=pl.BlockSpec((1, _SEQ // 128, 128), lambda i: (i, 0, 0)),
        out_shape=jax.ShapeDtypeStruct((n, _SEQ // 128, 128), jnp.int32),
    )(q3)


def _rank_body(ms_ref, lidx_ref, gp_ref, dest_scr):
    nr = ms_ref.shape[0]
    nc = _SEQ // 128
    m3 = ms_ref[...]
    m2d = m3.reshape(nr, _SEQ)

    cnt0 = jnp.sum((m2d >= 0).astype(jnp.int32), axis=1, keepdims=True)
    prefix0 = jnp.where(cnt0 >= _TOPK, jnp.int32(0), jnp.int32(_INT_MIN))

    def bit_body(b, prefix):
        bit = lax.shift_left(jnp.int32(1), jnp.int32(30) - b)
        cand = prefix | bit
        cnt = jnp.sum((m2d >= cand).astype(jnp.int32), axis=1, keepdims=True)
        return jnp.where(cnt >= _TOPK, cand, prefix)

    t = lax.fori_loop(0, 31, bit_body, prefix0)
    t3 = t[:, :, None]

    gt = m3 > t3
    eq = m3 == t3
    gtf = gt.astype(jnp.float32)
    eqf = eq.astype(jnp.float32)

    u128 = (lax.broadcasted_iota(jnp.int32, (128, 128), 0)
            < lax.broadcasted_iota(jnp.int32, (128, 128), 1)).astype(jnp.float32)
    u32s = (lax.broadcasted_iota(jnp.int32, (nc, nc), 0)
            < lax.broadcasted_iota(jnp.int32, (nc, nc), 1)).astype(jnp.float32)

    def ex_prefix(f3):
        lane = lax.dot_general(f3.reshape(nr * nc, 128), u128,
                               (((1,), (0,)), ((), ())),
                               preferred_element_type=jnp.float32)
        chs = jnp.sum(f3, axis=2)
        chpre = lax.dot_general(chs, u32s, (((1,), (0,)), ((), ())),
                                preferred_element_type=jnp.float32)
        return lane.reshape(nr, nc, 128) + chpre[:, :, None]

    c_gt = jnp.sum(jnp.sum(gtf, axis=2), axis=1)[:, None, None]
    r = jnp.float32(_TOPK) - c_gt
    peq = ex_prefix(eqf)
    sel = gt | (eq & (peq < r))
    psel = ex_prefix(sel.astype(jnp.float32))
    dest = jnp.where(sel, psel, jnp.float32(_TOPK)).astype(jnp.int32)
    dest_scr[...] = dest

    flatf = (lax.broadcasted_iota(jnp.int32, (nc, 128), 0) * 128
             + lax.broadcasted_iota(jnp.int32, (nc, 128), 1))
    tio = lax.broadcasted_iota(jnp.int32, (nc, 128, _TOPK), 2)

    def row_body(i, carry):
        d2 = dest_scr[pl.ds(i, 1)][0]
        e2 = (d2[:, :, None] == tio).astype(jnp.int32)
        contrib = e2 * flatf[:, :, None]
        inv = jnp.sum(jnp.sum(contrib, axis=0), axis=0)
        lidx_ref[pl.ds(i, 1), :] = inv.reshape(1, _TOPK)
        gp_ref[pl.ds(i, 1), :] = (
            (inv + jnp.int32(_SEQ) * i) >> 1).reshape(1, _TOPK)
        return carry

    lax.fori_loop(0, nr, row_body, jnp.int32(0))


def _run_rank(ms):
    n = ms.shape[0]
    return pl.pallas_call(
        _rank_body,
        out_shape=[
            jax.ShapeDtypeStruct((n, _TOPK), jnp.int32),
            jax.ShapeDtypeStruct((n, _TOPK), jnp.int32),
        ],
        scratch_shapes=[pltpu.VMEM((n, _SEQ // 128, 128), jnp.int32)],
    )(ms)


def _sc_body(gidx_hbm, q_hbm, k_hbm, v_hbm,
             qs_hbm, ks_hbm, vs_hbm,
             gidx_v, qs_v, ks_v, vs_v, sem):
    p = lax.axis_index("s") * 2 + lax.axis_index("c")

    pltpu.sync_copy(gidx_hbm.at[p], gidx_v)

    cpq = pltpu.async_copy(q_hbm.at[gidx_v], qs_v, sem)
    cpk = pltpu.async_copy(k_hbm.at[gidx_v], ks_v, sem)
    cpv = pltpu.async_copy(v_hbm.at[gidx_v], vs_v, sem)
    cpq.wait()
    cpk.wait()
    cpv.wait()

    pltpu.sync_copy(qs_v, qs_hbm.at[p])
    pltpu.sync_copy(ks_v, ks_hbm.at[p])
    pltpu.sync_copy(vs_v, vs_hbm.at[p])


def _run_select_gather(gidx2, q2, k2, v2):
    n = gidx2.shape[0]
    mesh = plsc.VectorSubcoreMesh(core_axis_name="c", subcore_axis_name="s")
    f = functools.partial(
        pl.kernel,
        mesh=mesh,
        out_type=[
            jax.ShapeDtypeStruct((n, _TOPK, 2 * _DK), jnp.float32),
            jax.ShapeDtypeStruct((n, _TOPK, 2 * _DK), jnp.float32),
            jax.ShapeDtypeStruct((n, _TOPK, 2 * _DK), jnp.float32),
        ],
        scratch_types=[
            pltpu.VMEM((_TOPK,), jnp.int32),
            pltpu.VMEM((_TOPK, 2 * _DK), jnp.float32),
            pltpu.VMEM((_TOPK, 2 * _DK), jnp.float32),
            pltpu.VMEM((_TOPK, 2 * _DK), jnp.float32),
            pltpu.SemaphoreType.DMA,
        ],
    )(_sc_body)
    return f(gidx2, q2, k2, v2)


def _attention_body(qs_ref, ks_ref, vs_ref, idx_ref, out_ref):
    local = idx_ref[0]
    par = (local & 1)[0][:, None] == 1

    def pick(ref):
        full = ref[0]
        return jnp.where(par, full[:, _DK:], full[:, :_DK])

    qb = pick(qs_ref)
    kb = pick(ks_ref)
    vb = pick(vs_ref)
    s = lax.dot_general(qb, kb, (((1,), (1,)), ((), ())),
                        preferred_element_type=jnp.float32)
    s = s * (1.0 / math.sqrt(_DK))
    mx = jnp.max(s, axis=-1, keepdims=True)
    e = jnp.exp(s - mx)
    w = e / jnp.sum(e, axis=-1, keepdims=True)
    att = lax.dot_general(w, vb, (((1,), (0,)), ((), ())),
                          preferred_element_type=jnp.float32)
    rows = lax.broadcasted_iota(jnp.int32, (_SEQ, _TOPK), 0)
    p = (rows == local).astype(jnp.bfloat16)
    a_hi = att.astype(jnp.bfloat16)
    a_lo = (att - a_hi.astype(jnp.float32)).astype(jnp.bfloat16)
    out = (lax.dot_general(p, a_hi, (((1,), (0,)), ((), ())),
                           preferred_element_type=jnp.float32)
           + lax.dot_general(p, a_lo, (((1,), (0,)), ((), ())),
                             preferred_element_type=jnp.float32))
    out_ref[0] = out.reshape(_SEQ // _DK, _DK, _DK)


def _run_attention(qs, ks, vs, idx3, batch):
    n = qs.shape[0]
    hpb = n // batch
    return pl.pallas_call(
        _attention_body,
        grid=(n,),
        in_specs=[
            pl.BlockSpec((1, _TOPK, 2 * _DK), lambda i: (i, 0, 0)),
            pl.BlockSpec((1, _TOPK, 2 * _DK), lambda i: (i, 0, 0)),
            pl.BlockSpec((1, _TOPK, 2 * _DK), lambda i: (i, 0, 0)),
            pl.BlockSpec((1, 1, _TOPK), lambda i: (i, 0, 0)),
        ],
        out_specs=pl.BlockSpec(
            (1, _SEQ // _DK, _DK, _DK),
            lambda i: (i // hpb, i % hpb, 0, 0)),
        out_shape=jax.ShapeDtypeStruct(
            (batch, hpb * (_SEQ // _DK), _DK, _DK), jnp.float32),
    )(qs, ks, vs, idx3)


def kernel(q, k, v):
    B, H, S, D = q.shape
    n = B * H
    q3 = q.reshape(n, S, D)
    k3 = k.reshape(n, S, D)
    v3 = v.reshape(n, S, D)

    ms = _run_importance(q3)
    lidx, gpair = _run_rank(ms)

    qs, ks, vs = _run_select_gather(
        gpair, q3.reshape(n * S // 2, 2 * D), k3.reshape(n * S // 2, 2 * D),
        v3.reshape(n * S // 2, 2 * D))

    return _run_attention(qs, ks, vs, lidx.reshape(n, 1, _TOPK), B)


# speedup vs baseline: 2.6483x; 1.2370x over previous
import functools
import math

import jax
import jax.numpy as jnp
from jax import lax
from jax.experimental import pallas as pl
from jax.experimental.pallas import tpu as pltpu
from jax.experimental.pallas import tpu_sc as plsc

_TOPK = 128
_SEQ = 4096
_DK = 64
_INT_MIN = -2147483648
_DUMP = 144


def _importance_body(q_ref, ms_ref, qp_ref):
    x = q_ref[0]
    mean = jnp.mean(x, axis=-1)
    xc = x - mean[:, None]
    var = jnp.sum(xc * xc, axis=-1) * (1.0 / (_DK - 1))
    imp = mean + jnp.sqrt(var)

    u = lax.bitcast_convert_type(imp, jnp.int32)
    ms = jnp.where(u >= 0, u, u ^ jnp.int32(0x7FFFFFFF))
    ms_ref[0] = ms.reshape(_SEQ // 128, 128)
    x3 = x.reshape(_SEQ // 2, 2, _DK)
    qp_ref[0] = jnp.concatenate([x3[:, 0, :], x3[:, 1, :]], axis=1)


def _run_importance(q3):
    n = q3.shape[0]
    return pl.pallas_call(
        _importance_body,
        grid=(n,),
        in_specs=[pl.BlockSpec((1, _SEQ, _DK), lambda i: (i, 0, 0))],
        out_specs=[
            pl.BlockSpec((1, _SEQ // 128, 128), lambda i: (i, 0, 0)),
            pl.BlockSpec((1, _SEQ // 2, 2 * _DK), lambda i: (i, 0, 0)),
        ],
        out_shape=[
            jax.ShapeDtypeStruct((n, _SEQ // 128, 128), jnp.int32),
            jax.ShapeDtypeStruct((n, _SEQ // 2, 2 * _DK), jnp.float32),
        ],
    )(q3)


def _rank_body(ms_ref, lidx_ref, gp_ref, dest_scr):
    nr = ms_ref.shape[0]
    nc = _SEQ // 128
    m3 = ms_ref[...]
    m2d = m3.reshape(nr, _SEQ)

    cnt0 = jnp.sum((m2d >= 0).astype(jnp.int32), axis=1, keepdims=True)
    prefix0 = jnp.where(cnt0 >= _TOPK, jnp.int32(0), jnp.int32(_INT_MIN))

    def bit_body(b, prefix):
        bit = lax.shift_left(jnp.int32(1), jnp.int32(30) - b)
        cand = prefix | bit
        cnt = jnp.sum((m2d >= cand).astype(jnp.int32), axis=1, keepdims=True)
        return jnp.where(cnt >= _TOPK, cand, prefix)

    t = lax.fori_loop(0, 31, bit_body, prefix0)
    t3 = t[:, :, None]

    gt = m3 > t3
    eq = m3 == t3
    gtf = gt.astype(jnp.float32)
    eqf = eq.astype(jnp.float32)

    u128 = (lax.broadcasted_iota(jnp.int32, (128, 128), 0)
            < lax.broadcasted_iota(jnp.int32, (128, 128), 1)).astype(jnp.float32)
    u32s = (lax.broadcasted_iota(jnp.int32, (nc, nc), 0)
            < lax.broadcasted_iota(jnp.int32, (nc, nc), 1)).astype(jnp.float32)

    def ex_prefix(f3):
        lane = lax.dot_general(f3.reshape(nr * nc, 128), u128,
                               (((1,), (0,)), ((), ())),
                               preferred_element_type=jnp.float32)
        chs = jnp.sum(f3, axis=2)
        chpre = lax.dot_general(chs, u32s, (((1,), (0,)), ((), ())),
                                preferred_element_type=jnp.float32)
        return lane.reshape(nr, nc, 128) + chpre[:, :, None]

    c_gt = jnp.sum(jnp.sum(gtf, axis=2), axis=1)[:, None, None]
    r = jnp.float32(_TOPK) - c_gt
    peq = ex_prefix(eqf)
    sel = gt | (eq & (peq < r))
    psel = ex_prefix(sel.astype(jnp.float32))
    dest = jnp.where(sel, psel, jnp.float32(_TOPK)).astype(jnp.int32)
    dest_scr[...] = dest

    flatf = (lax.broadcasted_iota(jnp.int32, (nc, 128), 0) * 128
             + lax.broadcasted_iota(jnp.int32, (nc, 128), 1))
    tio = lax.broadcasted_iota(jnp.int32, (nc, 128, _TOPK), 2)

    def row_body(i, carry):
        d2 = dest_scr[pl.ds(i, 1)][0]
        e2 = (d2[:, :, None] == tio).astype(jnp.int32)
        contrib = e2 * flatf[:, :, None]
        inv = jnp.sum(jnp.sum(contrib, axis=0), axis=0)
        lidx_ref[pl.ds(i, 1), :] = inv.reshape(1, _TOPK)
        gp_ref[pl.ds(i, 1), :] = (
            (inv + jnp.int32(_SEQ) * i) >> 1).reshape(1, _TOPK)
        return carry

    lax.fori_loop(0, nr, row_body, jnp.int32(0))


def _run_rank(ms):
    n = ms.shape[0]
    return pl.pallas_call(
        _rank_body,
        out_shape=[
            jax.ShapeDtypeStruct((n, _TOPK), jnp.int32),
            jax.ShapeDtypeStruct((n, _TOPK), jnp.int32),
        ],
        scratch_shapes=[pltpu.VMEM((n, _SEQ // 128, 128), jnp.int32)],
    )(ms)


def _sc_body(gidx_hbm, q_hbm, qs_hbm, gidx_v, qs_v, sem):
    p = lax.axis_index("s") * 2 + lax.axis_index("c")

    pltpu.sync_copy(gidx_hbm.at[p], gidx_v)

    pltpu.async_copy(q_hbm.at[gidx_v], qs_v, sem).wait()
    pltpu.sync_copy(qs_v, qs_hbm.at[p])


def _run_select_gather(gidx2, q2):
    n = gidx2.shape[0]
    mesh = plsc.VectorSubcoreMesh(core_axis_name="c", subcore_axis_name="s")
    f = functools.partial(
        pl.kernel,
        mesh=mesh,
        out_type=jax.ShapeDtypeStruct((n, _TOPK, 2 * _DK), jnp.float32),
        scratch_types=[
            pltpu.VMEM((_TOPK,), jnp.int32),
            pltpu.VMEM((_TOPK, 2 * _DK), jnp.float32),
            pltpu.SemaphoreType.DMA,
        ],
    )(_sc_body)
    return f(gidx2, q2)


def _attention_body(qs_ref, k_ref, v_ref, idx_ref, out_ref):
    local = idx_ref[0]
    par = (local & 1)[0][:, None] == 1

    qfull = qs_ref[0]
    qb = jnp.where(par, qfull[:, _DK:], qfull[:, :_DK])

    rows = lax.broadcasted_iota(jnp.int32, (_SEQ, _TOPK), 0)
    p = (rows == local).astype(jnp.bfloat16)

    def gather(ref):
        full = ref[0]
        hi = full.astype(jnp.bfloat16)
        lo = (full - hi.astype(jnp.float32)).astype(jnp.bfloat16)
        return (lax.dot_general(p, hi, (((0,), (0,)), ((), ())),
                                preferred_element_type=jnp.float32)
                + lax.dot_general(p, lo, (((0,), (0,)), ((), ())),
                                  preferred_element_type=jnp.float32))

    kb = gather(k_ref)
    vb = gather(v_ref)
    s = lax.dot_general(qb, kb, (((1,), (1,)), ((), ())),
                        preferred_element_type=jnp.float32)
    s = s * (1.0 / math.sqrt(_DK))
    mx = jnp.max(s, axis=-1, keepdims=True)
    e = jnp.exp(s - mx)
    w = e / jnp.sum(e, axis=-1, keepdims=True)
    att = lax.dot_general(w, vb, (((1,), (0,)), ((), ())),
                          preferred_element_type=jnp.float32)
    a_hi = att.astype(jnp.bfloat16)
    a_lo = (att - a_hi.astype(jnp.float32)).astype(jnp.bfloat16)
    out = (lax.dot_general(p, a_hi, (((1,), (0,)), ((), ())),
                           preferred_element_type=jnp.float32)
           + lax.dot_general(p, a_lo, (((1,), (0,)), ((), ())),
                             preferred_element_type=jnp.float32))
    out_ref[0] = out.reshape(_SEQ // _DK, _DK, _DK)


def _run_attention(qs, k3, v3, idx3, batch):
    n = qs.shape[0]
    hpb = n // batch
    return pl.pallas_call(
        _attention_body,
        grid=(n,),
        in_specs=[
            pl.BlockSpec((1, _TOPK, 2 * _DK), lambda i: (i, 0, 0)),
            pl.BlockSpec((1, _SEQ, _DK), lambda i: (i, 0, 0)),
            pl.BlockSpec((1, _SEQ, _DK), lambda i: (i, 0, 0)),
            pl.BlockSpec((1, 1, _TOPK), lambda i: (i, 0, 0)),
        ],
        out_specs=pl.BlockSpec(
            (1, _SEQ // _DK, _DK, _DK),
            lambda i: (i // hpb, i % hpb, 0, 0)),
        out_shape=jax.ShapeDtypeStruct(
            (batch, hpb * (_SEQ // _DK), _DK, _DK), jnp.float32),
    )(qs, k3, v3, idx3)


def kernel(q, k, v):
    B, H, S, D = q.shape
    n = B * H
    q3 = q.reshape(n, S, D)
    k3 = k.reshape(n, S, D)
    v3 = v.reshape(n, S, D)

    ms, qpack = _run_importance(q3)
    lidx, gpair = _run_rank(ms)

    qs = _run_select_gather(gpair, qpack.reshape(n * S // 2, 2 * D))

    return _run_attention(qs, k3, v3, lidx.reshape(n, 1, _TOPK), B)


# speedup vs baseline: 3.0346x; 1.1459x over previous
import functools
import math

import jax
import jax.numpy as jnp
from jax import lax
from jax.experimental import pallas as pl
from jax.experimental.pallas import tpu as pltpu
from jax.experimental.pallas import tpu_sc as plsc

_TOPK = 128
_SEQ = 4096
_DK = 64
_INT_MIN = -2147483648
_DUMP = 144


def _importance_body(q_ref, ms_ref, qp_ref):
    xt = q_ref[0]
    mean = jnp.mean(xt, axis=0)
    xc = xt - mean[None, :]
    var = jnp.sum(xc * xc, axis=0) * (1.0 / (_DK - 1))
    imp = mean + jnp.sqrt(var)

    u = lax.bitcast_convert_type(imp, jnp.int32)
    ms = jnp.where(u >= 0, u, u ^ jnp.int32(0x7FFFFFFF))
    ms_ref[0] = ms.reshape(_SEQ // 128, 128)
    eye = (lax.broadcasted_iota(jnp.int32, (_DK, _DK), 0)
           == lax.broadcasted_iota(jnp.int32, (_DK, _DK), 1)
           ).astype(jnp.float32)
    x = lax.dot_general(xt, eye, (((0,), (0,)), ((), ())),
                        preferred_element_type=jnp.float32)
    x3 = x.reshape(_SEQ // 2, 2, _DK)
    qp_ref[0] = jnp.concatenate([x3[:, 0, :], x3[:, 1, :]], axis=1)


def _run_importance(qt3):
    n = qt3.shape[0]
    return pl.pallas_call(
        _importance_body,
        grid=(n,),
        in_specs=[pl.BlockSpec((1, _DK, _SEQ), lambda i: (i, 0, 0))],
        out_specs=[
            pl.BlockSpec((1, _SEQ // 128, 128), lambda i: (i, 0, 0)),
            pl.BlockSpec((1, _SEQ // 2, 2 * _DK), lambda i: (i, 0, 0)),
        ],
        out_shape=[
            jax.ShapeDtypeStruct((n, _SEQ // 128, 128), jnp.int32),
            jax.ShapeDtypeStruct((n, _SEQ // 2, 2 * _DK), jnp.float32),
        ],
    )(qt3)


def _rank_body(ms_ref, lidx_ref, gp_ref, dest_scr):
    nr = ms_ref.shape[0]
    nc = _SEQ // 128
    m3 = ms_ref[...]
    m2d = m3.reshape(nr, _SEQ)

    cnt0 = jnp.sum((m2d >= 0).astype(jnp.int32), axis=1, keepdims=True)
    prefix0 = jnp.where(cnt0 >= _TOPK, jnp.int32(0), jnp.int32(_INT_MIN))

    def bit_body(b, prefix):
        bit = lax.shift_left(jnp.int32(1), jnp.int32(30) - b)
        cand = prefix | bit
        cnt = jnp.sum((m2d >= cand).astype(jnp.int32), axis=1, keepdims=True)
        return jnp.where(cnt >= _TOPK, cand, prefix)

    t = lax.fori_loop(0, 31, bit_body, prefix0)
    t3 = t[:, :, None]

    gt = m3 > t3
    eq = m3 == t3
    gtf = gt.astype(jnp.float32)
    eqf = eq.astype(jnp.float32)

    u128 = (lax.broadcasted_iota(jnp.int32, (128, 128), 0)
            < lax.broadcasted_iota(jnp.int32, (128, 128), 1)).astype(jnp.float32)
    u32s = (lax.broadcasted_iota(jnp.int32, (nc, nc), 0)
            < lax.broadcasted_iota(jnp.int32, (nc, nc), 1)).astype(jnp.float32)

    def ex_prefix(f3):
        lane = lax.dot_general(f3.reshape(nr * nc, 128), u128,
                               (((1,), (0,)), ((), ())),
                               preferred_element_type=jnp.float32)
        chs = jnp.sum(f3, axis=2)
        chpre = lax.dot_general(chs, u32s, (((1,), (0,)), ((), ())),
                                preferred_element_type=jnp.float32)
        return lane.reshape(nr, nc, 128) + chpre[:, :, None]

    c_gt = jnp.sum(jnp.sum(gtf, axis=2), axis=1)[:, None, None]
    r = jnp.float32(_TOPK) - c_gt
    peq = ex_prefix(eqf)
    sel = gt | (eq & (peq < r))
    psel = ex_prefix(sel.astype(jnp.float32))
    dest = jnp.where(sel, psel, jnp.float32(_TOPK)).astype(jnp.int32)
    dest_scr[...] = dest

    tio = lax.broadcasted_iota(jnp.int32, (nc, 128, _TOPK), 2)
    flatf = (lax.broadcasted_iota(jnp.int32, (nc, 128), 0) * 128
             + lax.broadcasted_iota(jnp.int32, (nc, 128), 1))

    def row_body(i, carry):
        d2 = dest_scr[pl.ds(i, 1)][0]
        e2 = (d2[:, :, None] == tio).astype(jnp.int32)
        contrib = e2 * flatf[:, :, None]
        inv = jnp.sum(jnp.sum(contrib, axis=0), axis=0)
        lidx_ref[pl.ds(i, 1), :] = inv.reshape(1, _TOPK)
        gp_ref[pl.ds(i, 1), :] = (
            (inv + jnp.int32(_SEQ) * i) >> 1).reshape(1, _TOPK)
        return carry

    lax.fori_loop(0, nr, row_body, jnp.int32(0))


def _run_rank(ms):
    n = ms.shape[0]
    return pl.pallas_call(
        _rank_body,
        out_shape=[
            jax.ShapeDtypeStruct((n, _TOPK), jnp.int32),
            jax.ShapeDtypeStruct((n, _TOPK), jnp.int32),
        ],
        scratch_shapes=[pltpu.VMEM((n, _SEQ // 128, 128), jnp.int32)],
    )(ms)


def _sc_body(gidx_hbm, q_hbm, qs_hbm, gidx_v, qs_v, sem):
    p = lax.axis_index("s") * 2 + lax.axis_index("c")

    pltpu.sync_copy(gidx_hbm.at[p], gidx_v)

    pltpu.async_copy(q_hbm.at[gidx_v], qs_v, sem).wait()
    pltpu.sync_copy(qs_v, qs_hbm.at[p])


def _run_select_gather(gidx2, q2):
    n = gidx2.shape[0]
    mesh = plsc.VectorSubcoreMesh(core_axis_name="c", subcore_axis_name="s")
    f = functools.partial(
        pl.kernel,
        mesh=mesh,
        out_type=jax.ShapeDtypeStruct((n, _TOPK, 2 * _DK), jnp.float32),
        scratch_types=[
            pltpu.VMEM((_TOPK,), jnp.int32),
            pltpu.VMEM((_TOPK, 2 * _DK), jnp.float32),
            pltpu.SemaphoreType.DMA,
        ],
    )(_sc_body)
    return f(gidx2, q2)


def _attention_body(qs_ref, kt_ref, vt_ref, idx_ref, out_ref):
    local = idx_ref[0]
    par = (local & 1)[0][:, None] == 1

    qfull = qs_ref[0]
    qb = jnp.where(par, qfull[:, _DK:], qfull[:, :_DK])

    p = (lax.broadcasted_iota(jnp.int32, (_SEQ, _TOPK), 0)
         == local).astype(jnp.bfloat16)
    pt = (local[0][:, None]
          == lax.broadcasted_iota(jnp.int32, (_TOPK, _SEQ), 1)
          ).astype(jnp.bfloat16)

    def gather_cols(ref):
        full = ref[0]
        hi = full.astype(jnp.bfloat16)
        lo = (full - hi.astype(jnp.float32)).astype(jnp.bfloat16)
        return (lax.dot_general(hi, p, (((1,), (0,)), ((), ())),
                                preferred_element_type=jnp.float32)
                + lax.dot_general(lo, p, (((1,), (0,)), ((), ())),
                                  preferred_element_type=jnp.float32))

    ktg = gather_cols(kt_ref)
    vtg = gather_cols(vt_ref)
    s = lax.dot_general(qb, ktg, (((1,), (0,)), ((), ())),
                        preferred_element_type=jnp.float32)
    s = s * (1.0 / math.sqrt(_DK))
    mx = jnp.max(s, axis=-1, keepdims=True)
    e = jnp.exp(s - mx)
    w = e / jnp.sum(e, axis=-1, keepdims=True)
    att_t = lax.dot_general(vtg, w, (((1,), (1,)), ((), ())),
                            preferred_element_type=jnp.float32)
    a_hi = att_t.astype(jnp.bfloat16)
    a_lo = (att_t - a_hi.astype(jnp.float32)).astype(jnp.bfloat16)
    out = (lax.dot_general(a_hi, pt, (((1,), (0,)), ((), ())),
                           preferred_element_type=jnp.float32)
           + lax.dot_general(a_lo, pt, (((1,), (0,)), ((), ())),
                             preferred_element_type=jnp.float32))
    out_ref[0] = out


def _run_attention(qs, kt3, vt3, idx3):
    n = qs.shape[0]
    return pl.pallas_call(
        _attention_body,
        grid=(n,),
        in_specs=[
            pl.BlockSpec((1, _TOPK, 2 * _DK), lambda i: (i, 0, 0)),
            pl.BlockSpec((1, _DK, _SEQ), lambda i: (i, 0, 0)),
            pl.BlockSpec((1, _DK, _SEQ), lambda i: (i, 0, 0)),
            pl.BlockSpec((1, 1, _TOPK), lambda i: (i, 0, 0)),
        ],
        out_specs=pl.BlockSpec((1, _DK, _SEQ), lambda i: (i, 0, 0)),
        out_shape=jax.ShapeDtypeStruct((n, _DK, _SEQ), jnp.float32),
    )(qs, kt3, vt3, idx3)


def kernel(q, k, v):
    B, H, S, D = q.shape
    n = B * H
    qt3 = jnp.transpose(q, (0, 1, 3, 2)).reshape(n, D, S)
    kt3 = jnp.transpose(k, (0, 1, 3, 2)).reshape(n, D, S)
    vt3 = jnp.transpose(v, (0, 1, 3, 2)).reshape(n, D, S)

    ms, qpack = _run_importance(qt3)
    lidx, gpair = _run_rank(ms)

    qs = _run_select_gather(gpair, qpack.reshape(n * S // 2, 2 * D))

    out_t = _run_attention(qs, kt3, vt3, lidx.reshape(n, 1, _TOPK))
    hw = int(math.sqrt(S))
    out = jnp.transpose(out_t.reshape(B, H, D, S), (0, 1, 3, 2))
    return out.reshape(B, -1, hw, hw)


# speedup vs baseline: 3.6447x; 1.2011x over previous
import functools
import math

import jax
import jax.numpy as jnp
from jax import lax
from jax.experimental import pallas as pl
from jax.experimental.pallas import tpu as pltpu
from jax.experimental.pallas import tpu_sc as plsc

_TOPK = 128
_SEQ = 4096
_DK = 64
_INT_MIN = -2147483648
_DUMP = 144


def _importance_body(q_ref, ms_ref, qp_ref):
    xt = q_ref[0]
    mean = jnp.mean(xt, axis=0)
    xc = xt - mean[None, :]
    var = jnp.sum(xc * xc, axis=0) * (1.0 / (_DK - 1))
    imp = mean + jnp.sqrt(var)

    u = lax.bitcast_convert_type(imp, jnp.int32)
    ms = jnp.where(u >= 0, u, u ^ jnp.int32(0x7FFFFFFF))
    ms_ref[0] = ms.reshape(_SEQ // 128, 128)
    eye = (lax.broadcasted_iota(jnp.int32, (_DK, _DK), 0)
           == lax.broadcasted_iota(jnp.int32, (_DK, _DK), 1)
           ).astype(jnp.float32)
    x = lax.dot_general(xt, eye, (((0,), (0,)), ((), ())),
                        preferred_element_type=jnp.float32)
    x3 = x.reshape(_SEQ // 2, 2, _DK)
    qp_ref[0] = jnp.concatenate([x3[:, 0, :], x3[:, 1, :]], axis=1)


def _run_importance(qt3):
    n = qt3.shape[0]
    return pl.pallas_call(
        _importance_body,
        grid=(n,),
        in_specs=[pl.BlockSpec((1, _DK, _SEQ), lambda i: (i, 0, 0))],
        out_specs=[
            pl.BlockSpec((1, _SEQ // 128, 128), lambda i: (i, 0, 0)),
            pl.BlockSpec((1, _SEQ // 2, 2 * _DK), lambda i: (i, 0, 0)),
        ],
        out_shape=[
            jax.ShapeDtypeStruct((n, _SEQ // 128, 128), jnp.int32),
            jax.ShapeDtypeStruct((n, _SEQ // 2, 2 * _DK), jnp.float32),
        ],
    )(qt3)


def _rank_body(ms_ref, lidx_ref, gp_ref, dest_scr):
    nr = ms_ref.shape[0]
    nc = _SEQ // 128
    m3 = ms_ref[...]
    m2d = m3.reshape(nr, _SEQ)

    cnt0 = jnp.sum((m2d >= 0).astype(jnp.int32), axis=1, keepdims=True)
    prefix0 = jnp.where(cnt0 >= _TOPK, jnp.int32(0), jnp.int32(_INT_MIN))

    def bit_body(b, prefix):
        bit = lax.shift_left(jnp.int32(1), jnp.int32(30) - b)
        cand = prefix | bit
        cnt = jnp.sum((m2d >= cand).astype(jnp.int32), axis=1, keepdims=True)
        return jnp.where(cnt >= _TOPK, cand, prefix)

    t = lax.fori_loop(0, 31, bit_body, prefix0)
    t3 = t[:, :, None]

    gt = m3 > t3
    eq = m3 == t3
    gtf = gt.astype(jnp.float32)
    eqf = eq.astype(jnp.float32)

    u128 = (lax.broadcasted_iota(jnp.int32, (128, 128), 0)
            < lax.broadcasted_iota(jnp.int32, (128, 128), 1)).astype(jnp.float32)
    u32s = (lax.broadcasted_iota(jnp.int32, (nc, nc), 0)
            < lax.broadcasted_iota(jnp.int32, (nc, nc), 1)).astype(jnp.float32)

    def ex_prefix(f3):
        lane = lax.dot_general(f3.reshape(nr * nc, 128), u128,
                               (((1,), (0,)), ((), ())),
                               preferred_element_type=jnp.float32)
        chs = jnp.sum(f3, axis=2)
        chpre = lax.dot_general(chs, u32s, (((1,), (0,)), ((), ())),
                                preferred_element_type=jnp.float32)
        return lane.reshape(nr, nc, 128) + chpre[:, :, None]

    c_gt = jnp.sum(jnp.sum(gtf, axis=2), axis=1)[:, None, None]
    r = jnp.float32(_TOPK) - c_gt
    peq = ex_prefix(eqf)
    sel = gt | (eq & (peq < r))
    psel = ex_prefix(sel.astype(jnp.float32))
    dest = jnp.where(sel, psel, jnp.float32(_TOPK)).astype(jnp.int32)
    dest_scr[...] = dest

    tio = lax.broadcasted_iota(jnp.int32, (nc, 128, _TOPK), 2)
    flatf = (lax.broadcasted_iota(jnp.int32, (nc, 128), 0) * 128
             + lax.broadcasted_iota(jnp.int32, (nc, 128), 1))

    def row_body(i, carry):
        d2 = dest_scr[pl.ds(i, 1)][0]
        e2 = (d2[:, :, None] == tio).astype(jnp.int32)
        contrib = e2 * flatf[:, :, None]
        inv = jnp.sum(jnp.sum(contrib, axis=0), axis=0)
        lidx_ref[pl.ds(i, 1), :] = inv.reshape(1, _TOPK)
        gp_ref[pl.ds(i, 1), :] = (
            (inv + jnp.int32(_SEQ) * i) >> 1).reshape(1, _TOPK)
        return carry

    lax.fori_loop(0, nr, row_body, jnp.int32(0))


def _run_rank(ms):
    n = ms.shape[0]
    return pl.pallas_call(
        _rank_body,
        out_shape=[
            jax.ShapeDtypeStruct((n, _TOPK), jnp.int32),
            jax.ShapeDtypeStruct((n, _TOPK), jnp.int32),
        ],
        scratch_shapes=[pltpu.VMEM((n, _SEQ // 128, 128), jnp.int32)],
    )(ms)


def _sc_body(gidx_hbm, q_hbm, qs_hbm, gidx_v, qs_v, sem):
    p = lax.axis_index("s") * 2 + lax.axis_index("c")

    pltpu.sync_copy(gidx_hbm.at[p], gidx_v)

    pltpu.async_copy(q_hbm.at[gidx_v], qs_v, sem).wait()
    pltpu.sync_copy(qs_v, qs_hbm.at[p])


def _run_select_gather(gidx2, q2):
    n = gidx2.shape[0]
    mesh = plsc.VectorSubcoreMesh(core_axis_name="c", subcore_axis_name="s")
    f = functools.partial(
        pl.kernel,
        mesh=mesh,
        out_type=jax.ShapeDtypeStruct((n, _TOPK, 2 * _DK), jnp.float32),
        scratch_types=[
            pltpu.VMEM((_TOPK,), jnp.int32),
            pltpu.VMEM((_TOPK, 2 * _DK), jnp.float32),
            pltpu.SemaphoreType.DMA,
        ],
    )(_sc_body)
    return f(gidx2, q2)


def _attention_body(qs_ref, kt_ref, vt_ref, idx_ref, out_ref):
    local = idx_ref[0]
    par = (local & 1)[0][:, None] == 1

    qfull = qs_ref[0]
    qb = jnp.where(par, qfull[:, _DK:], qfull[:, :_DK])

    p = (lax.broadcasted_iota(jnp.int32, (_SEQ, _TOPK), 0)
         == local).astype(jnp.bfloat16)

    def gather_cols(ref):
        full = ref[0]
        hi = full.astype(jnp.bfloat16)
        lo = (full - hi.astype(jnp.float32)).astype(jnp.bfloat16)
        return (lax.dot_general(hi, p, (((1,), (0,)), ((), ())),
                                preferred_element_type=jnp.float32)
                + lax.dot_general(lo, p, (((1,), (0,)), ((), ())),
                                  preferred_element_type=jnp.float32))

    ktg = gather_cols(kt_ref)
    vtg = gather_cols(vt_ref)
    s = lax.dot_general(qb, ktg, (((1,), (0,)), ((), ())),
                        preferred_element_type=jnp.float32)
    s = s * (1.0 / math.sqrt(_DK))
    mx = jnp.max(s, axis=-1, keepdims=True)
    e = jnp.exp(s - mx)
    w = e / jnp.sum(e, axis=-1, keepdims=True)
    att = lax.dot_general(w, vtg, (((1,), (1,)), ((), ())),
                          preferred_element_type=jnp.float32)
    a_hi = att.astype(jnp.bfloat16)
    a_lo = (att - a_hi.astype(jnp.float32)).astype(jnp.bfloat16)
    out = (lax.dot_general(p, a_hi, (((1,), (0,)), ((), ())),
                           preferred_element_type=jnp.float32)
           + lax.dot_general(p, a_lo, (((1,), (0,)), ((), ())),
                             preferred_element_type=jnp.float32))
    out_ref[0] = out.reshape(_SEQ // _DK, _DK, _DK)


def _run_attention(qs, kt3, vt3, idx3, batch):
    n = qs.shape[0]
    hpb = n // batch
    return pl.pallas_call(
        _attention_body,
        grid=(n,),
        in_specs=[
            pl.BlockSpec((1, _TOPK, 2 * _DK), lambda i: (i, 0, 0)),
            pl.BlockSpec((1, _DK, _SEQ), lambda i: (i, 0, 0)),
            pl.BlockSpec((1, _DK, _SEQ), lambda i: (i, 0, 0)),
            pl.BlockSpec((1, 1, _TOPK), lambda i: (i, 0, 0)),
        ],
        out_specs=pl.BlockSpec(
            (1, _SEQ // _DK, _DK, _DK),
            lambda i: (i // hpb, i % hpb, 0, 0)),
        out_shape=jax.ShapeDtypeStruct(
            (batch, hpb * (_SEQ // _DK), _DK, _DK), jnp.float32),
    )(qs, kt3, vt3, idx3)


def kernel(q, k, v):
    B, H, S, D = q.shape
    n = B * H
    qt3 = jnp.transpose(q, (0, 1, 3, 2)).reshape(n, D, S)
    kt3 = jnp.transpose(k, (0, 1, 3, 2)).reshape(n, D, S)
    vt3 = jnp.transpose(v, (0, 1, 3, 2)).reshape(n, D, S)

    ms, qpack = _run_importance(qt3)
    lidx, gpair = _run_rank(ms)

    qs = _run_select_gather(gpair, qpack.reshape(n * S // 2, 2 * D))

    return _run_attention(qs, kt3, vt3, lidx.reshape(n, 1, _TOPK), B)
